# Initial kernel scaffold; baseline (speedup 1.0000x reference)
import jax
import jax.numpy as jnp
from jax.experimental import pallas as pl


def kernel(u, i, j, b_tm1, VUI, VIU, VIL, VLI):
    raise NotImplementedError("write your pallas kernel here")


# speedup vs baseline: 1.1239x; 1.1239x over previous
import functools

import jax
import jax.numpy as jnp
from jax import lax
from jax.experimental import pallas as pl
from jax.experimental.pallas import tpu as pltpu
from jax.experimental.pallas import tpu_sc as plsc

NC = 2
NS = 16
LANES = 16
F = 128
NCHUNK = F // LANES
L_BASKET = 200
ROWS_PER_TILE = 16
PAD = NS * ROWS_PER_TILE
N_BASKET_TILES = 13
INV_L = 1.0 / L_BASKET

_mesh = plsc.VectorSubcoreMesh(
    core_axis_name="c", subcore_axis_name="s", num_cores=NC, num_subcores=NS
)


@functools.partial(
    pl.kernel,
    out_type=jax.ShapeDtypeStruct((LANES,), jnp.float32),
    mesh=_mesh,
    scratch_types=[
        pltpu.VMEM((ROWS_PER_TILE,), jnp.int32),
        pltpu.VMEM((ROWS_PER_TILE, F), jnp.float32),
        pltpu.VMEM((F,), jnp.float32),
        pltpu.VMEM((8,), jnp.int32),
        pltpu.VMEM((8,), jnp.int32),
        pltpu.VMEM((8, F), jnp.float32),
        pltpu.VMEM((8, F), jnp.float32),
        pltpu.VMEM((LANES,), jnp.float32),
        pltpu.VMEM((F,), jnp.float32),
        pltpu.VMEM((N_BASKET_TILES, F), jnp.float32),
        pltpu.VMEM((LANES,), jnp.float32),
        pltpu.VMEM_SHARED((N_BASKET_TILES, F), jnp.float32),
        pltpu.VMEM_SHARED((LANES,), jnp.float32),
        pltpu.VMEM_SHARED((F,), jnp.float32),
        pltpu.SemaphoreType.DMA,
    ],
)
def _fpmc_sc(idx_hbm, ij_hbm, u_hbm, vui_hbm, viu_hbm, vil_hbm, vli_hbm, out_hbm,
             idx_v, rows_v, acc_v, ij_v, u_v, rows_a, rows_b,
             sv_v, dil_v, bsum_v, out_v,
             shared_basket, shared_sv, shared_dil, sem):
    c = lax.axis_index("c")
    s = lax.axis_index("s")

    @pl.when(jnp.logical_and(c == 0, s < N_BASKET_TILES))
    def _basket():
        base = s * ROWS_PER_TILE
        pltpu.sync_copy(idx_hbm.at[pl.ds(base, ROWS_PER_TILE)], idx_v)
        pltpu.async_copy(vli_hbm.at[idx_v], rows_v, sem).wait()
        accs = [jnp.zeros((LANES,), jnp.float32) for _ in range(NCHUNK)]
        for r in range(ROWS_PER_TILE):
            w = jnp.where(base + r < L_BASKET, 1.0, 0.0).astype(jnp.float32)
            for k in range(NCHUNK):
                accs[k] = accs[k] + rows_v[r, pl.ds(k * LANES, LANES)] * w
        for k in range(NCHUNK):
            acc_v[pl.ds(k * LANES, LANES)] = accs[k]
        pltpu.sync_copy(acc_v, shared_basket.at[s])

    @pl.when(jnp.logical_and(c == 0, s == N_BASKET_TILES))
    def _user_item():
        pltpu.sync_copy(ij_hbm, ij_v)
        pltpu.sync_copy(u_hbm, u_v)
        pltpu.async_copy(vui_hbm.at[u_v], rows_a, sem).wait()
        pltpu.async_copy(viu_hbm.at[ij_v], rows_b, sem).wait()
        sv = jnp.zeros((LANES,), jnp.float32)
        for k in range(NCHUNK):
            dsl = pl.ds(k * LANES, LANES)
            sv = sv + rows_a[0, dsl] * (rows_b[0, dsl] - rows_b[1, dsl])
        sv_v[...] = sv
        pltpu.sync_copy(sv_v, shared_sv)

    @pl.when(jnp.logical_and(c == 0, s == N_BASKET_TILES + 1))
    def _item_diff():
        pltpu.sync_copy(ij_hbm, ij_v)
        pltpu.async_copy(vil_hbm.at[ij_v], rows_a, sem).wait()
        for k in range(NCHUNK):
            dsl = pl.ds(k * LANES, LANES)
            dil_v[dsl] = rows_a[0, dsl] - rows_a[1, dsl]
        pltpu.sync_copy(dil_v, shared_dil)

    plsc.subcore_barrier()

    @pl.when(jnp.logical_and(c == 0, s == 0))
    def _combine():
        pltpu.sync_copy(shared_basket, bsum_v)
        pltpu.sync_copy(shared_sv, sv_v)
        pltpu.sync_copy(shared_dil, dil_v)
        tot = sv_v[...]
        for k in range(NCHUNK):
            dsl = pl.ds(k * LANES, LANES)
            m = bsum_v[0, dsl]
            for t in range(1, N_BASKET_TILES):
                m = m + bsum_v[t, dsl]
            tot = tot + dil_v[dsl] * (m * INV_L)
        d = tot[0]
        for k in range(1, LANES):
            d = d + tot[k]
        db = jnp.full((LANES,), d, dtype=jnp.float32)
        out_v[...] = 1.0 / (1.0 + jnp.exp(db))
        pltpu.sync_copy(out_v, out_hbm)


def kernel(u, i, j, b_tm1, VUI, VIU, VIL, VLI):
    b = b_tm1.astype(jnp.int32)
    idx = jnp.zeros((PAD,), jnp.int32).at[:L_BASKET].set(b)
    ii = jnp.asarray(i, jnp.int32)
    jj = jnp.asarray(j, jnp.int32)
    ij8 = jnp.tile(jnp.stack([ii, jj]), 4)
    u8 = jnp.full((8,), jnp.asarray(u, jnp.int32))
    out = _fpmc_sc(idx, ij8, u8, VUI, VIU, VIL, VLI)
    return out[0]


# speedup vs baseline: 1.2442x; 1.1070x over previous
import functools

import jax
import jax.numpy as jnp
from jax import lax
from jax.experimental import pallas as pl
from jax.experimental.pallas import tpu as pltpu
from jax.experimental.pallas import tpu_sc as plsc

NC = 2
NS = 16
LANES = 16
F = 128
NCHUNK = F // LANES
L_BASKET = 200
ROWS_PER_TILE = 16
N_FULL_TILES = L_BASKET // ROWS_PER_TILE
TAIL_ROWS = L_BASKET - N_FULL_TILES * ROWS_PER_TILE
TILE_TAIL = N_FULL_TILES
TILE_UI = TILE_TAIL + 1
TILE_IL = TILE_UI + 1
N_PARTIALS = TILE_IL + 1
INV_L = 1.0 / L_BASKET

_mesh = plsc.VectorSubcoreMesh(
    core_axis_name="c", subcore_axis_name="s", num_cores=NC, num_subcores=NS
)


@functools.partial(
    pl.kernel,
    out_type=jax.ShapeDtypeStruct((LANES,), jnp.float32),
    mesh=_mesh,
    scratch_types=[
        pltpu.VMEM((ROWS_PER_TILE,), jnp.int32),
        pltpu.VMEM((ROWS_PER_TILE, F), jnp.float32),
        pltpu.VMEM((F,), jnp.float32),
        pltpu.VMEM((8,), jnp.int32),
        pltpu.VMEM((8,), jnp.int32),
        pltpu.VMEM((2, F), jnp.float32),
        pltpu.VMEM((2, F), jnp.float32),
        pltpu.VMEM((LANES,), jnp.float32),
        pltpu.VMEM((N_PARTIALS, F), jnp.float32),
        pltpu.VMEM((LANES,), jnp.float32),
        pltpu.VMEM_SHARED((N_PARTIALS, F), jnp.float32),
        pltpu.SemaphoreType.DMA,
    ],
)
def _fpmc_sc(idx_hbm, ij_hbm, u_hbm, vui_hbm, viu_hbm, vil_hbm, vli_hbm, out_hbm,
             idx_v, rows_v, acc_v, ij_v, u_v, rows_a, rows_b,
             sv_v, buf_v, out_v, shared, sem):
    c = lax.axis_index("c")
    s = lax.axis_index("s")

    def column_sum(nrows):
        accs = [rows_v[0, pl.ds(k * LANES, LANES)] for k in range(NCHUNK)]
        for r in range(1, nrows):
            for k in range(NCHUNK):
                accs[k] = accs[k] + rows_v[r, pl.ds(k * LANES, LANES)]
        for k in range(NCHUNK):
            acc_v[pl.ds(k * LANES, LANES)] = accs[k]
        pltpu.sync_copy(acc_v, shared.at[s])

    @pl.when(jnp.logical_and(c == 0, s < N_FULL_TILES))
    def _basket_full():
        pltpu.sync_copy(idx_hbm.at[pl.ds(s * ROWS_PER_TILE, ROWS_PER_TILE)], idx_v)
        pltpu.async_copy(vli_hbm.at[idx_v], rows_v, sem).wait()
        column_sum(ROWS_PER_TILE)

    @pl.when(jnp.logical_and(c == 0, s == TILE_TAIL))
    def _basket_tail():
        pltpu.sync_copy(
            idx_hbm.at[pl.ds(N_FULL_TILES * ROWS_PER_TILE, TAIL_ROWS)],
            idx_v.at[pl.ds(0, TAIL_ROWS)],
        )
        pltpu.async_copy(
            vli_hbm.at[idx_v.at[pl.ds(0, TAIL_ROWS)]],
            rows_v.at[pl.ds(0, TAIL_ROWS)],
            sem,
        ).wait()
        column_sum(TAIL_ROWS)

    @pl.when(jnp.logical_and(c == 0, s == TILE_UI))
    def _user_item():
        pltpu.sync_copy(ij_hbm, ij_v)
        pltpu.sync_copy(u_hbm, u_v)
        ca = pltpu.async_copy(vui_hbm.at[u_v.at[pl.ds(0, 2)]], rows_a, sem)
        cb = pltpu.async_copy(viu_hbm.at[ij_v.at[pl.ds(0, 2)]], rows_b, sem)
        ca.wait()
        cb.wait()
        sv = jnp.zeros((LANES,), jnp.float32)
        for k in range(NCHUNK):
            dsl = pl.ds(k * LANES, LANES)
            sv = sv + rows_a[0, dsl] * (rows_b[0, dsl] - rows_b[1, dsl])
        sv_v[...] = sv
        pltpu.sync_copy(sv_v, shared.at[TILE_UI, pl.ds(0, LANES)])

    @pl.when(jnp.logical_and(c == 0, s == TILE_IL))
    def _item_diff():
        pltpu.sync_copy(ij_hbm, ij_v)
        pltpu.async_copy(vil_hbm.at[ij_v.at[pl.ds(0, 2)]], rows_a, sem).wait()
        for k in range(NCHUNK):
            dsl = pl.ds(k * LANES, LANES)
            acc_v[dsl] = rows_a[0, dsl] - rows_a[1, dsl]
        pltpu.sync_copy(acc_v, shared.at[TILE_IL])

    plsc.subcore_barrier()

    @pl.when(jnp.logical_and(c == 0, s == 0))
    def _combine():
        pltpu.sync_copy(shared, buf_v)
        tot = buf_v[TILE_UI, pl.ds(0, LANES)]
        for k in range(NCHUNK):
            dsl = pl.ds(k * LANES, LANES)
            m = buf_v[0, dsl]
            for t in range(1, N_FULL_TILES + 1):
                m = m + buf_v[t, dsl]
            tot = tot + buf_v[TILE_IL, dsl] * (m * INV_L)
        d = tot[0]
        for k in range(1, LANES):
            d = d + tot[k]
        db = jnp.full((LANES,), d, dtype=jnp.float32)
        out_v[...] = 1.0 / (1.0 + jnp.exp(db))
        pltpu.sync_copy(out_v, out_hbm)


def kernel(u, i, j, b_tm1, VUI, VIU, VIL, VLI):
    idx = b_tm1.astype(jnp.int32)
    ii = jnp.asarray(i, jnp.int32)
    jj = jnp.asarray(j, jnp.int32)
    ij8 = jnp.tile(jnp.stack([ii, jj]), 4)
    u8 = jnp.full((8,), jnp.asarray(u, jnp.int32))
    out = _fpmc_sc(idx, ij8, u8, VUI, VIU, VIL, VLI)
    return out[0]


# speedup vs baseline: 1.3159x; 1.0577x over previous
import functools

import jax
import jax.numpy as jnp
from jax import lax
from jax.experimental import pallas as pl
from jax.experimental.pallas import tpu as pltpu
from jax.experimental.pallas import tpu_sc as plsc

NC = 2
NS = 16
LANES = 16
F = 128
NCHUNK = F // LANES
L_BASKET = 200
ROWS_PER_TILE = 16
N_FULL_TILES = L_BASKET // ROWS_PER_TILE
TAIL_ROWS = L_BASKET - N_FULL_TILES * ROWS_PER_TILE
TILE_TAIL = N_FULL_TILES
TILE_UI = TILE_TAIL + 1
TILE_IL = TILE_UI + 1
N_PARTIALS = TILE_IL + 1
INV_L = 1.0 / L_BASKET
OFF_IJ = 208
OFF_U = 216
IDX_LEN = 224

_mesh = plsc.VectorSubcoreMesh(
    core_axis_name="c", subcore_axis_name="s", num_cores=NC, num_subcores=NS
)


@functools.partial(
    pl.kernel,
    out_type=jax.ShapeDtypeStruct((LANES,), jnp.float32),
    mesh=_mesh,
    scratch_types=[
        pltpu.VMEM((ROWS_PER_TILE,), jnp.int32),
        pltpu.VMEM((ROWS_PER_TILE, F), jnp.float32),
        pltpu.VMEM((F,), jnp.float32),
        pltpu.VMEM((8,), jnp.int32),
        pltpu.VMEM((8,), jnp.int32),
        pltpu.VMEM((2, F), jnp.float32),
        pltpu.VMEM((2, F), jnp.float32),
        pltpu.VMEM((LANES,), jnp.float32),
        pltpu.VMEM((N_PARTIALS, F), jnp.float32),
        pltpu.VMEM((LANES,), jnp.float32),
        pltpu.VMEM_SHARED((N_PARTIALS, F), jnp.float32),
        pltpu.SemaphoreType.DMA,
    ],
)
def _fpmc_sc(idx_hbm, vui_hbm, viu_hbm, vil_hbm, vli_hbm, out_hbm,
             idx_v, rows_v, acc_v, ij_v, u_v, rows_a, rows_b,
             sv_v, buf_v, out_v, shared, sem):
    c = lax.axis_index("c")
    s = lax.axis_index("s")

    def column_sum(nrows):
        accs = [rows_v[0, pl.ds(k * LANES, LANES)] for k in range(NCHUNK)]
        for r in range(1, nrows):
            for k in range(NCHUNK):
                accs[k] = accs[k] + rows_v[r, pl.ds(k * LANES, LANES)]
        for k in range(NCHUNK):
            acc_v[pl.ds(k * LANES, LANES)] = accs[k]
        pltpu.sync_copy(acc_v, shared.at[s])

    @pl.when(jnp.logical_and(c == 0, s < N_FULL_TILES))
    def _basket_full():
        pltpu.sync_copy(idx_hbm.at[pl.ds(s * ROWS_PER_TILE, ROWS_PER_TILE)], idx_v)
        pltpu.async_copy(vli_hbm.at[idx_v], rows_v, sem).wait()
        column_sum(ROWS_PER_TILE)

    @pl.when(jnp.logical_and(c == 0, s == TILE_TAIL))
    def _basket_tail():
        pltpu.sync_copy(
            idx_hbm.at[pl.ds(N_FULL_TILES * ROWS_PER_TILE, TAIL_ROWS)],
            idx_v.at[pl.ds(0, TAIL_ROWS)],
        )
        pltpu.async_copy(
            vli_hbm.at[idx_v.at[pl.ds(0, TAIL_ROWS)]],
            rows_v.at[pl.ds(0, TAIL_ROWS)],
            sem,
        ).wait()
        column_sum(TAIL_ROWS)

    @pl.when(jnp.logical_and(c == 0, s == TILE_UI))
    def _user_item():
        pltpu.sync_copy(idx_hbm.at[pl.ds(OFF_IJ, 8)], ij_v)
        pltpu.sync_copy(idx_hbm.at[pl.ds(OFF_U, 8)], u_v)
        ca = pltpu.async_copy(vui_hbm.at[u_v.at[pl.ds(0, 2)]], rows_a, sem)
        cb = pltpu.async_copy(viu_hbm.at[ij_v.at[pl.ds(0, 2)]], rows_b, sem)
        ca.wait()
        cb.wait()
        sv = jnp.zeros((LANES,), jnp.float32)
        for k in range(NCHUNK):
            dsl = pl.ds(k * LANES, LANES)
            sv = sv + rows_a[0, dsl] * (rows_b[0, dsl] - rows_b[1, dsl])
        sv_v[...] = sv
        pltpu.sync_copy(sv_v, shared.at[TILE_UI, pl.ds(0, LANES)])

    @pl.when(jnp.logical_and(c == 0, s == TILE_IL))
    def _item_diff():
        pltpu.sync_copy(idx_hbm.at[pl.ds(OFF_IJ, 8)], ij_v)
        pltpu.async_copy(vil_hbm.at[ij_v.at[pl.ds(0, 2)]], rows_a, sem).wait()
        for k in range(NCHUNK):
            dsl = pl.ds(k * LANES, LANES)
            acc_v[dsl] = rows_a[0, dsl] - rows_a[1, dsl]
        pltpu.sync_copy(acc_v, shared.at[TILE_IL])

    plsc.subcore_barrier()

    @pl.when(jnp.logical_and(c == 0, s == 0))
    def _combine():
        pltpu.sync_copy(shared, buf_v)
        tot = buf_v[TILE_UI, pl.ds(0, LANES)]
        for k in range(NCHUNK):
            dsl = pl.ds(k * LANES, LANES)
            m = buf_v[0, dsl]
            for t in range(1, N_FULL_TILES + 1):
                m = m + buf_v[t, dsl]
            tot = tot + buf_v[TILE_IL, dsl] * (m * INV_L)
        d = tot[0]
        for k in range(1, LANES):
            d = d + tot[k]
        db = jnp.full((LANES,), d, dtype=jnp.float32)
        out_v[...] = 1.0 / (1.0 + jnp.exp(db))
        pltpu.sync_copy(out_v, out_hbm)


def kernel(u, i, j, b_tm1, VUI, VIU, VIL, VLI):
    ii = jnp.asarray(i, jnp.int32)
    jj = jnp.asarray(j, jnp.int32)
    uu = jnp.asarray(u, jnp.int32)
    z = jnp.zeros((1,), jnp.int32)
    idx = jnp.concatenate([
        b_tm1.astype(jnp.int32),
        jnp.zeros((8,), jnp.int32),
        ii[None], jj[None], z, z, z, z, z, z,
        uu[None], uu[None], z, z, z, z, z, z,
    ])
    out = _fpmc_sc(idx, VUI, VIU, VIL, VLI)
    return out[0]


# speedup vs baseline: 1.3274x; 1.0087x over previous
import functools

import jax
import jax.numpy as jnp
from jax import lax
from jax.experimental import pallas as pl
from jax.experimental.pallas import tpu as pltpu
from jax.experimental.pallas import tpu_sc as plsc

NC = 2
NS = 16
LANES = 16
F = 128
NCHUNK = F // LANES
L_BASKET = 200
ROWS_PER_TILE = 16
N_FULL_TILES = L_BASKET // ROWS_PER_TILE
TAIL_ROWS = L_BASKET - N_FULL_TILES * ROWS_PER_TILE
TILE_TAIL = N_FULL_TILES
TILE_UI = TILE_TAIL + 1
TILE_IL = TILE_UI + 1
N_PARTIALS = TILE_IL + 1
INV_L = 1.0 / L_BASKET
OFF_IJ = 208
OFF_U = 216
IDX_LEN = 224

_mesh = plsc.VectorSubcoreMesh(
    core_axis_name="c", subcore_axis_name="s", num_cores=1, num_subcores=NS
)


@functools.partial(
    pl.kernel,
    out_type=jax.ShapeDtypeStruct((LANES,), jnp.float32),
    mesh=_mesh,
    scratch_types=[
        pltpu.VMEM((ROWS_PER_TILE,), jnp.int32),
        pltpu.VMEM((ROWS_PER_TILE, F), jnp.float32),
        pltpu.VMEM((F,), jnp.float32),
        pltpu.VMEM((8,), jnp.int32),
        pltpu.VMEM((8,), jnp.int32),
        pltpu.VMEM((2, F), jnp.float32),
        pltpu.VMEM((2, F), jnp.float32),
        pltpu.VMEM((LANES,), jnp.float32),
        pltpu.VMEM((N_PARTIALS, F), jnp.float32),
        pltpu.VMEM((LANES,), jnp.float32),
        pltpu.VMEM_SHARED((N_PARTIALS, F), jnp.float32),
        pltpu.SemaphoreType.DMA,
    ],
)
def _fpmc_sc(idx_hbm, vui_hbm, viu_hbm, vil_hbm, vli_hbm, out_hbm,
             idx_v, rows_v, acc_v, ij_v, u_v, rows_a, rows_b,
             sv_v, buf_v, out_v, shared, sem):
    c = lax.axis_index("c")
    s = lax.axis_index("s")

    def column_sum(nrows):
        accs = [rows_v[0, pl.ds(k * LANES, LANES)] for k in range(NCHUNK)]
        for r in range(1, nrows):
            for k in range(NCHUNK):
                accs[k] = accs[k] + rows_v[r, pl.ds(k * LANES, LANES)]
        for k in range(NCHUNK):
            acc_v[pl.ds(k * LANES, LANES)] = accs[k]
        pltpu.sync_copy(acc_v, shared.at[s])

    @pl.when(jnp.logical_and(c == 0, s < N_FULL_TILES))
    def _basket_full():
        pltpu.sync_copy(idx_hbm.at[pl.ds(s * ROWS_PER_TILE, ROWS_PER_TILE)], idx_v)
        pltpu.async_copy(vli_hbm.at[idx_v], rows_v, sem).wait()
        column_sum(ROWS_PER_TILE)

    @pl.when(jnp.logical_and(c == 0, s == TILE_TAIL))
    def _basket_tail():
        pltpu.sync_copy(
            idx_hbm.at[pl.ds(N_FULL_TILES * ROWS_PER_TILE, TAIL_ROWS)],
            idx_v.at[pl.ds(0, TAIL_ROWS)],
        )
        pltpu.async_copy(
            vli_hbm.at[idx_v.at[pl.ds(0, TAIL_ROWS)]],
            rows_v.at[pl.ds(0, TAIL_ROWS)],
            sem,
        ).wait()
        column_sum(TAIL_ROWS)

    @pl.when(jnp.logical_and(c == 0, s == TILE_UI))
    def _user_item():
        pltpu.sync_copy(idx_hbm.at[pl.ds(OFF_IJ, 8)], ij_v)
        pltpu.sync_copy(idx_hbm.at[pl.ds(OFF_U, 8)], u_v)
        ca = pltpu.async_copy(vui_hbm.at[u_v.at[pl.ds(0, 2)]], rows_a, sem)
        cb = pltpu.async_copy(viu_hbm.at[ij_v.at[pl.ds(0, 2)]], rows_b, sem)
        ca.wait()
        cb.wait()
        sv = jnp.zeros((LANES,), jnp.float32)
        for k in range(NCHUNK):
            dsl = pl.ds(k * LANES, LANES)
            sv = sv + rows_a[0, dsl] * (rows_b[0, dsl] - rows_b[1, dsl])
        sv_v[...] = sv
        pltpu.sync_copy(sv_v, shared.at[TILE_UI, pl.ds(0, LANES)])

    @pl.when(jnp.logical_and(c == 0, s == TILE_IL))
    def _item_diff():
        pltpu.sync_copy(idx_hbm.at[pl.ds(OFF_IJ, 8)], ij_v)
        pltpu.async_copy(vil_hbm.at[ij_v.at[pl.ds(0, 2)]], rows_a, sem).wait()
        for k in range(NCHUNK):
            dsl = pl.ds(k * LANES, LANES)
            acc_v[dsl] = rows_a[0, dsl] - rows_a[1, dsl]
        pltpu.sync_copy(acc_v, shared.at[TILE_IL])

    plsc.subcore_barrier()

    @pl.when(jnp.logical_and(c == 0, s == 0))
    def _combine():
        pltpu.sync_copy(shared, buf_v)
        tot = buf_v[TILE_UI, pl.ds(0, LANES)]
        for k in range(NCHUNK):
            dsl = pl.ds(k * LANES, LANES)
            m = buf_v[0, dsl]
            for t in range(1, N_FULL_TILES + 1):
                m = m + buf_v[t, dsl]
            tot = tot + buf_v[TILE_IL, dsl] * (m * INV_L)
        d = tot[0]
        for k in range(1, LANES):
            d = d + tot[k]
        db = jnp.full((LANES,), d, dtype=jnp.float32)
        out_v[...] = 1.0 / (1.0 + jnp.exp(db))
        pltpu.sync_copy(out_v, out_hbm)


def kernel(u, i, j, b_tm1, VUI, VIU, VIL, VLI):
    ii = jnp.asarray(i, jnp.int32)
    jj = jnp.asarray(j, jnp.int32)
    uu = jnp.asarray(u, jnp.int32)
    z = jnp.zeros((1,), jnp.int32)
    idx = jnp.concatenate([
        b_tm1.astype(jnp.int32),
        jnp.zeros((8,), jnp.int32),
        ii[None], jj[None], z, z, z, z, z, z,
        uu[None], uu[None], z, z, z, z, z, z,
    ])
    out = _fpmc_sc(idx, VUI, VIU, VIL, VLI)
    return out[0]
